# baseline (speedup 1.0000x reference)
import jax
import jax.numpy as jnp
from jax import lax
from jax.experimental import pallas as pl
from jax.experimental.pallas import tpu as pltpu
from jax.experimental.pallas import tpu_sc as plsc

DX = (0.5, 0.5)
BX = (-49.75, -49.75)
BEV_DIM = (200, 200)

B = 256
T = 1800
L = 16
NC = 2
NS = 16
HB = 128
NSLAB = 16
TSTEP = 112
TLEN = 120
NG = HB // L
W0 = 88
WW = 24


def _sc_body(tr_hbm, c_hbm, out_hbm, cc_v, trx_v, try_v, out_v,
             ix_v, iy_v, row_v, sem_c, sem_c2, sem_x, sem_y):
    wid = lax.axis_index("s") * NC + lax.axis_index("c")
    slab = wid % NSLAB
    half = wid // NSLAB
    t0 = slab * TSTEP
    col0 = half * HB

    inv_dx = float(1.0 / DX[0])
    neg_bx = float(-BX[0])
    iota = lax.iota(jnp.int32, L)

    HW = WW // 2
    ccopy0 = pltpu.make_async_copy(
        c_hbm.at[pl.ds(W0, HW), pl.ds(W0, WW), pl.ds(col0, HB)],
        cc_v.at[pl.ds(0, HW)], sem_c)
    ccopy1 = pltpu.make_async_copy(
        c_hbm.at[pl.ds(W0 + HW, HW), pl.ds(W0, WW), pl.ds(col0, HB)],
        cc_v.at[pl.ds(HW, HW)], sem_c2)
    ccopy0.start()
    ccopy1.start()

    for j in range(NG):
        base = 16 * j if j < NG - 1 else TLEN - L
        tvec = t0 + base + iota
        ix_v[pl.ds(base, L)] = 4 * tvec + 2 * half
        iy_v[pl.ds(base, L)] = 4 * tvec + 2 * half + 1
    xcopy = pltpu.make_async_copy(tr_hbm.at[ix_v], trx_v, sem_x)
    ycopy = pltpu.make_async_copy(tr_hbm.at[iy_v], try_v, sem_y)
    xcopy.start()
    ycopy.start()
    xcopy.wait()
    ycopy.wait()
    ccopy0.wait()
    ccopy1.wait()

    def discretize(xx, yy):
        xi = jnp.clip(((xx + neg_bx) * inv_dx).astype(jnp.int32),
                      0, BEV_DIM[0] - 1)
        yi = jnp.clip(((yy + neg_bx) * inv_dx).astype(jnp.int32),
                      0, BEV_DIM[1] - 1)
        return xi, yi

    wmax = jnp.uint32(WW - 1)

    def step(tl, acc):
        for g in range(NG):
            xx = trx_v[tl, pl.ds(g * L, L)]
            yy = try_v[tl, pl.ds(g * L, L)]
            xs = ((xx + neg_bx) * inv_dx).astype(jnp.int32) - W0
            ys = ((yy + neg_bx) * inv_dx).astype(jnp.int32) - W0
            xw = plsc.bitcast(
                jnp.minimum(plsc.bitcast(xs, jnp.uint32), wmax), jnp.int32)
            yw = plsc.bitcast(
                jnp.minimum(plsc.bitcast(ys, jnp.uint32), wmax), jnp.int32)
            out_v[tl, pl.ds(g * L, L)] = plsc.load_gather(
                cc_v, [xw, yw, g * L + iota])
            acc = acc | (xs ^ xw) | (ys ^ yw)
        return acc

    tlen = jnp.where(slab == NSLAB - 1, TLEN, TSTEP)
    acc = lax.fori_loop(0, tlen, step, jnp.zeros((L,), jnp.int32))

    @pl.when(jnp.max(jnp.where(acc != 0, 1, 0)) > 0)
    def _fix():
        def fstep(tl, carry):
            for g in range(NG):
                xi, yi = discretize(trx_v[tl, pl.ds(g * L, L)],
                                    try_v[tl, pl.ds(g * L, L)])
                inwin = ((xi >= W0) & (xi < W0 + WW)
                         & (yi >= W0) & (yi < W0 + WW))
                miss = jnp.where(inwin, 0, 1)

                @pl.when(jnp.max(miss) > 0)
                def _group():
                    for l in range(L):
                        sel = jnp.where(iota == l, 1, 0)
                        m_l = jnp.sum(sel * miss)

                        @pl.when(m_l > 0)
                        def _lane():
                            xs = jnp.sum(sel * xi)
                            ys = jnp.sum(sel * yi)
                            pltpu.sync_copy(c_hbm.at[xs, ys], row_v)
                            val = plsc.load_gather(
                                row_v,
                                [jnp.full((L,), col0 + g * L + l, jnp.int32)])
                            plsc.store_scatter(
                                out_v,
                                [jnp.full((L,), tl, jnp.int32),
                                 jnp.full((L,), g * L + l, jnp.int32)],
                                val)
            return carry

        lax.fori_loop(0, tlen, fstep, 0)

    @pl.when(slab == NSLAB - 1)
    def _tail():
        pltpu.sync_copy(out_v, out_hbm.at[pl.ds(t0, TLEN), pl.ds(col0, HB)])

    @pl.when(slab != NSLAB - 1)
    def _body():
        pltpu.sync_copy(out_v.at[pl.ds(0, TSTEP)],
                        out_hbm.at[pl.ds(t0, TSTEP), pl.ds(col0, HB)])


@jax.jit
def kernel(trajs, C):
    c3 = jnp.transpose(C, (1, 2, 0))
    tr = jnp.transpose(trajs.reshape(2, HB, T, 2), (2, 0, 3, 1))
    tr = tr.reshape(4 * T, HB)
    run = pl.kernel(
        _sc_body,
        out_type=jax.ShapeDtypeStruct((T, B), jnp.float32),
        mesh=plsc.VectorSubcoreMesh(
            core_axis_name="c", subcore_axis_name="s",
            num_cores=NC, num_subcores=NS),
        scratch_types=[
            pltpu.VMEM((WW, WW, HB), jnp.float32),
            pltpu.VMEM((TLEN, HB), jnp.float32),
            pltpu.VMEM((TLEN, HB), jnp.float32),
            pltpu.VMEM((TLEN, HB), jnp.float32),
            pltpu.VMEM((TLEN,), jnp.int32),
            pltpu.VMEM((TLEN,), jnp.int32),
            pltpu.VMEM((B,), jnp.float32),
            pltpu.SemaphoreType.DMA,
            pltpu.SemaphoreType.DMA,
            pltpu.SemaphoreType.DMA,
            pltpu.SemaphoreType.DMA,
        ],
        compiler_params=pltpu.CompilerParams(
            needs_layout_passes=False, use_tc_tiling_on_sc=True),
    )
    return jnp.transpose(run(tr, c3))


# speedup vs baseline: 1.2731x; 1.2731x over previous
import jax
import jax.numpy as jnp
from jax import lax
from jax.experimental import pallas as pl
from jax.experimental.pallas import tpu as pltpu
from jax.experimental.pallas import tpu_sc as plsc

DX = (0.5, 0.5)
BX = (-49.75, -49.75)
BEV_DIM = (200, 200)

B = 256
T = 1800
L = 16
NC = 2
NS = 16
HB = 128
NSLAB = 16
TSTEP = 112
TLEN = 120
NG = HB // L
W0 = 88
WW = 24


def _sc_body(tr_hbm, c_hbm, out_hbm, cc_v, trx_v, try_v, out_v,
             ix_v, iy_v, row_v, sem_c, sem_c2, sem_x, sem_y):
    wid = lax.axis_index("s") * NC + lax.axis_index("c")
    slab = wid % NSLAB
    half = wid // NSLAB
    t0 = slab * TSTEP
    col0 = half * HB

    inv_dx = float(1.0 / DX[0])
    neg_bx = float(-BX[0])
    iota = lax.iota(jnp.int32, L)

    HW = WW // 2
    ccopy0 = pltpu.make_async_copy(
        c_hbm.at[pl.ds(W0, HW), pl.ds(W0, WW), pl.ds(col0, HB)],
        cc_v.at[pl.ds(0, HW)], sem_c)
    ccopy1 = pltpu.make_async_copy(
        c_hbm.at[pl.ds(W0 + HW, HW), pl.ds(W0, WW), pl.ds(col0, HB)],
        cc_v.at[pl.ds(HW, HW)], sem_c2)
    ccopy0.start()
    ccopy1.start()

    for j in range(NG):
        base = 16 * j if j < NG - 1 else TLEN - L
        tvec = t0 + base + iota
        ix_v[pl.ds(base, L)] = 4 * tvec + 2 * half
        iy_v[pl.ds(base, L)] = 4 * tvec + 2 * half + 1
    xcopy = pltpu.make_async_copy(tr_hbm.at[ix_v], trx_v, sem_x)
    ycopy = pltpu.make_async_copy(tr_hbm.at[iy_v], try_v, sem_y)
    xcopy.start()
    ycopy.start()
    xcopy.wait()
    ycopy.wait()
    ccopy0.wait()
    ccopy1.wait()

    def discretize(xx, yy):
        xi = jnp.clip(((xx + neg_bx) * inv_dx).astype(jnp.int32),
                      0, BEV_DIM[0] - 1)
        yi = jnp.clip(((yy + neg_bx) * inv_dx).astype(jnp.int32),
                      0, BEV_DIM[1] - 1)
        return xi, yi

    wmax = jnp.uint32(WW - 1)

    def step(tl, acc):
        for g in range(NG):
            xx = trx_v[tl, pl.ds(g * L, L)]
            yy = try_v[tl, pl.ds(g * L, L)]
            xs = ((xx + neg_bx) * inv_dx).astype(jnp.int32) - W0
            ys = ((yy + neg_bx) * inv_dx).astype(jnp.int32) - W0
            xw = plsc.bitcast(
                jnp.minimum(plsc.bitcast(xs, jnp.uint32), wmax), jnp.int32)
            yw = plsc.bitcast(
                jnp.minimum(plsc.bitcast(ys, jnp.uint32), wmax), jnp.int32)
            out_v[tl, pl.ds(g * L, L)] = plsc.load_gather(
                cc_v, [xw, yw, g * L + iota])
            acc = acc | (xs ^ xw) | (ys ^ yw)
        return acc

    acc = lax.fori_loop(0, TLEN, step, jnp.zeros((L,), jnp.int32))

    @pl.when(jnp.max(jnp.where(acc != 0, 1, 0)) > 0)
    def _fix():
        def fstep(tl, carry):
            for g in range(NG):
                xi, yi = discretize(trx_v[tl, pl.ds(g * L, L)],
                                    try_v[tl, pl.ds(g * L, L)])
                inwin = ((xi >= W0) & (xi < W0 + WW)
                         & (yi >= W0) & (yi < W0 + WW))
                miss = jnp.where(inwin, 0, 1)

                @pl.when(jnp.max(miss) > 0)
                def _group():
                    for l in range(L):
                        sel = jnp.where(iota == l, 1, 0)
                        m_l = jnp.sum(sel * miss)

                        @pl.when(m_l > 0)
                        def _lane():
                            xs = jnp.sum(sel * xi)
                            ys = jnp.sum(sel * yi)
                            pltpu.sync_copy(c_hbm.at[xs, ys], row_v)
                            val = plsc.load_gather(
                                row_v,
                                [jnp.full((L,), col0 + g * L + l, jnp.int32)])
                            plsc.store_scatter(
                                out_v,
                                [jnp.full((L,), tl, jnp.int32),
                                 jnp.full((L,), g * L + l, jnp.int32)],
                                val)
            return carry

        lax.fori_loop(0, TLEN, fstep, 0)

    pltpu.sync_copy(out_v, out_hbm.at[pl.ds(t0, TLEN), pl.ds(col0, HB)])


@jax.jit
def kernel(trajs, C):
    c3 = jnp.transpose(C, (1, 2, 0))
    tr = jnp.transpose(trajs.reshape(2, HB, T, 2), (2, 0, 3, 1))
    tr = tr.reshape(4 * T, HB)
    run = pl.kernel(
        _sc_body,
        out_type=jax.ShapeDtypeStruct((T, B), jnp.float32),
        mesh=plsc.VectorSubcoreMesh(
            core_axis_name="c", subcore_axis_name="s",
            num_cores=NC, num_subcores=NS),
        scratch_types=[
            pltpu.VMEM((WW, WW, HB), jnp.float32),
            pltpu.VMEM((TLEN, HB), jnp.float32),
            pltpu.VMEM((TLEN, HB), jnp.float32),
            pltpu.VMEM((TLEN, HB), jnp.float32),
            pltpu.VMEM((TLEN,), jnp.int32),
            pltpu.VMEM((TLEN,), jnp.int32),
            pltpu.VMEM((B,), jnp.float32),
            pltpu.SemaphoreType.DMA,
            pltpu.SemaphoreType.DMA,
            pltpu.SemaphoreType.DMA,
            pltpu.SemaphoreType.DMA,
        ],
        compiler_params=pltpu.CompilerParams(
            needs_layout_passes=False, use_tc_tiling_on_sc=True),
    )
    return jnp.transpose(run(tr, c3))


# speedup vs baseline: 1.2861x; 1.0102x over previous
import jax
import jax.numpy as jnp
from jax import lax
from jax.experimental import pallas as pl
from jax.experimental.pallas import tpu as pltpu
from jax.experimental.pallas import tpu_sc as plsc

DX = (0.5, 0.5)
BX = (-49.75, -49.75)
BEV_DIM = (200, 200)

B = 256
T = 1800
L = 16
NC = 2
NS = 16
HB = 128
NSLAB = 16
TSTEP = 112
TLEN = 120
NG = HB // L
W0 = 88
WW = 24


def _sc_body(tr_hbm, c_hbm, out_hbm, cc_v, trx_v, try_v, out_v,
             ix_v, iy_v, row_v, sem_c, sem_x, sem_y):
    wid = lax.axis_index("s") * NC + lax.axis_index("c")
    slab = wid % NSLAB
    half = wid // NSLAB
    t0 = slab * TSTEP
    col0 = half * HB

    inv_dx = float(1.0 / DX[0])
    neg_bx = float(-BX[0])
    iota = lax.iota(jnp.int32, L)

    ccopy = pltpu.make_async_copy(
        c_hbm.at[pl.ds(W0, WW), pl.ds(W0, WW), pl.ds(col0, HB)], cc_v, sem_c)
    ccopy.start()

    for j in range(NG):
        base = 16 * j if j < NG - 1 else TLEN - L
        tvec = t0 + base + iota
        ix_v[pl.ds(base, L)] = 4 * tvec + 2 * half
        iy_v[pl.ds(base, L)] = 4 * tvec + 2 * half + 1
    xcopy = pltpu.make_async_copy(tr_hbm.at[ix_v], trx_v, sem_x)
    ycopy = pltpu.make_async_copy(tr_hbm.at[iy_v], try_v, sem_y)
    xcopy.start()
    ycopy.start()
    xcopy.wait()
    ycopy.wait()
    ccopy.wait()

    def discretize(xx, yy):
        xi = jnp.clip(((xx + neg_bx) * inv_dx).astype(jnp.int32),
                      0, BEV_DIM[0] - 1)
        yi = jnp.clip(((yy + neg_bx) * inv_dx).astype(jnp.int32),
                      0, BEV_DIM[1] - 1)
        return xi, yi

    wmax = jnp.uint32(WW - 1)

    def step(tl, acc):
        for g in range(NG):
            xx = trx_v[tl, pl.ds(g * L, L)]
            yy = try_v[tl, pl.ds(g * L, L)]
            xs = ((xx + neg_bx) * inv_dx).astype(jnp.int32) - W0
            ys = ((yy + neg_bx) * inv_dx).astype(jnp.int32) - W0
            xw = plsc.bitcast(
                jnp.minimum(plsc.bitcast(xs, jnp.uint32), wmax), jnp.int32)
            yw = plsc.bitcast(
                jnp.minimum(plsc.bitcast(ys, jnp.uint32), wmax), jnp.int32)
            out_v[tl, pl.ds(g * L, L)] = plsc.load_gather(
                cc_v, [xw, yw, g * L + iota])
            acc = acc | (xs ^ xw) | (ys ^ yw)
        return acc

    acc = lax.fori_loop(0, TLEN, step, jnp.zeros((L,), jnp.int32))

    @pl.when(jnp.max(jnp.where(acc != 0, 1, 0)) > 0)
    def _fix():
        def fstep(tl, carry):
            for g in range(NG):
                xi, yi = discretize(trx_v[tl, pl.ds(g * L, L)],
                                    try_v[tl, pl.ds(g * L, L)])
                inwin = ((xi >= W0) & (xi < W0 + WW)
                         & (yi >= W0) & (yi < W0 + WW))
                miss = jnp.where(inwin, 0, 1)

                @pl.when(jnp.max(miss) > 0)
                def _group():
                    for l in range(L):
                        sel = jnp.where(iota == l, 1, 0)
                        m_l = jnp.sum(sel * miss)

                        @pl.when(m_l > 0)
                        def _lane():
                            xs = jnp.sum(sel * xi)
                            ys = jnp.sum(sel * yi)
                            pltpu.sync_copy(c_hbm.at[xs, ys], row_v)
                            val = plsc.load_gather(
                                row_v,
                                [jnp.full((L,), col0 + g * L + l, jnp.int32)])
                            plsc.store_scatter(
                                out_v,
                                [jnp.full((L,), tl, jnp.int32),
                                 jnp.full((L,), g * L + l, jnp.int32)],
                                val)
            return carry

        lax.fori_loop(0, TLEN, fstep, 0)

    pltpu.sync_copy(out_v, out_hbm.at[pl.ds(t0, TLEN), pl.ds(col0, HB)])


@jax.jit
def kernel(trajs, C):
    c3 = jnp.transpose(C, (1, 2, 0))
    tr = jnp.transpose(trajs.reshape(2, HB, T, 2), (2, 0, 3, 1))
    tr = tr.reshape(4 * T, HB)
    run = pl.kernel(
        _sc_body,
        out_type=jax.ShapeDtypeStruct((T, B), jnp.float32),
        mesh=plsc.VectorSubcoreMesh(
            core_axis_name="c", subcore_axis_name="s",
            num_cores=NC, num_subcores=NS),
        scratch_types=[
            pltpu.VMEM((WW, WW, HB), jnp.float32),
            pltpu.VMEM((TLEN, HB), jnp.float32),
            pltpu.VMEM((TLEN, HB), jnp.float32),
            pltpu.VMEM((TLEN, HB), jnp.float32),
            pltpu.VMEM((TLEN,), jnp.int32),
            pltpu.VMEM((TLEN,), jnp.int32),
            pltpu.VMEM((B,), jnp.float32),
            pltpu.SemaphoreType.DMA,
            pltpu.SemaphoreType.DMA,
            pltpu.SemaphoreType.DMA,
        ],
        compiler_params=pltpu.CompilerParams(
            needs_layout_passes=False, use_tc_tiling_on_sc=True,
            skip_device_barrier=True),
    )
    return jnp.transpose(run(tr, c3))
